# baseline (speedup 1.0000x reference)
import jax
import jax.numpy as jnp
from jax.experimental import pallas as pl
from jax.experimental.pallas import tpu as pltpu

DIM = 4096
INTER = 14336
T = 32

C1 = 128
NC1 = INTER // C1
NB1 = 10
C2 = 64
NC2 = DIM // C2
NB2 = 6


def _ffn_kernel(xt_ref, w1_hbm, w2_hbm, o_ref, w1buf, w2buf, h_ref, sem1, sem2):
    def start1(c):
        b = jax.lax.rem(c, NB1)
        pltpu.make_async_copy(
            w1_hbm.at[pl.ds(c * C1, C1), :], w1buf.at[b], sem1.at[b]).start()

    def start2(c):
        b = jax.lax.rem(c, NB2)
        pltpu.make_async_copy(
            w2_hbm.at[pl.ds(c * C2, C2), :], w2buf.at[b], sem2.at[b]).start()

    for c in range(NB1):
        start1(c)

    def body1(c, carry):
        b = jax.lax.rem(c, NB1)
        pltpu.make_async_copy(
            w1_hbm.at[pl.ds(c * C1, C1), :], w1buf.at[b], sem1.at[b]).wait()
        h = jax.lax.dot_general(
            w1buf[b], xt_ref[...],
            dimension_numbers=(((1,), (0,)), ((), ())),
            preferred_element_type=jnp.float32,
        )
        h_ref[pl.ds(c * C1, C1), :] = jnp.maximum(h, 0.0)

        @pl.when(c + NB1 < NC1)
        def _():
            start1(c + NB1)

        @pl.when((c + NB1 >= NC1) & (c + NB1 < NC1 + NB2))
        def _():
            start2(c + NB1 - NC1)

        return carry

    jax.lax.fori_loop(0, NC1, body1, 0)

    def body2(c, carry):
        b = jax.lax.rem(c, NB2)
        pltpu.make_async_copy(
            w2_hbm.at[pl.ds(c * C2, C2), :], w2buf.at[b], sem2.at[b]).wait()
        o_ref[pl.ds(c * C2, C2), :] = jax.lax.dot_general(
            w2buf[b], h_ref[...],
            dimension_numbers=(((1,), (0,)), ((), ())),
            preferred_element_type=jnp.float32,
        )

        @pl.when(c + NB2 < NC2)
        def _():
            start2(c + NB2)

        return carry

    jax.lax.fori_loop(0, NC2, body2, 0)


@jax.jit
def kernel(x, W1, W2):
    b, t, d = x.shape
    xt = x.reshape(b * t, d).T
    out_t = pl.pallas_call(
        _ffn_kernel,
        in_specs=[
            pl.BlockSpec(memory_space=pltpu.MemorySpace.VMEM),
            pl.BlockSpec(memory_space=pltpu.MemorySpace.HBM),
            pl.BlockSpec(memory_space=pltpu.MemorySpace.HBM),
        ],
        out_specs=pl.BlockSpec(memory_space=pltpu.MemorySpace.VMEM),
        out_shape=jax.ShapeDtypeStruct((DIM, T), jnp.float32),
        scratch_shapes=[
            pltpu.MemorySpace.VMEM((NB1, C1, DIM), jnp.float32),
            pltpu.MemorySpace.VMEM((NB2, C2, INTER), jnp.float32),
            pltpu.MemorySpace.VMEM((INTER, T), jnp.float32),
            pltpu.SemaphoreType.DMA((NB1,)),
            pltpu.SemaphoreType.DMA((NB2,)),
        ],
    )(xt, W1, W2)
    return out_t.T.reshape(b, t, d)
